# baseline (speedup 1.0000x reference)
import functools

import numpy as np
import jax
import jax.numpy as jnp
from jax import lax
from jax.experimental import pallas as pl
from jax.experimental.pallas import tpu as pltpu
from jax.experimental.pallas import tpu_sc as plsc

_B = 1024
_S = 200
_D = 64
_N = _B * _S
_NC = 2
_NS = 16
_NW = _NC * _NS
_PER_W = _N // _NW
_CH = 128
_CHUNKS = _PER_W // _CH
_LANES = 16


def _pe_doubled() -> np.ndarray:
    pos = np.arange(_S, dtype=np.float32)[:, None]
    div = 10000.0 ** (np.arange(0, _D, 2, dtype=np.float32) / _D)
    pe = np.zeros((_S, _D), dtype=np.float32)
    pe[:, 0::2] = np.sin(pos / div)
    pe[:, 1::2] = np.cos(pos / div)
    return np.concatenate([pe, pe], axis=0)


_PE2 = _pe_doubled()

_mesh = plsc.VectorSubcoreMesh(core_axis_name="c", subcore_axis_name="s")


@functools.partial(
    pl.kernel,
    mesh=_mesh,
    out_type=jax.ShapeDtypeStruct((_N, _D), jnp.float32),
    compiler_params=pltpu.CompilerParams(use_tc_tiling_on_sc=False),
    scratch_types=[
        pltpu.VMEM((_PER_W,), jnp.int32),
        pltpu.VMEM((2 * _S, _D), jnp.float32),
        pltpu.VMEM((_CH, _D), jnp.float32),
        pltpu.SemaphoreType.DMA,
    ],
)
def _emb_pe(idx_hbm, table_hbm, pe2_hbm, out_hbm, idx_v, pe2_v, buf, sem):
    wid = lax.axis_index("s") * _NC + lax.axis_index("c")
    base = wid * _PER_W
    pltpu.sync_copy(idx_hbm.at[pl.ds(base, _PER_W)], idx_v)
    pltpu.sync_copy(pe2_hbm, pe2_v)

    def chunk_body(c, carry):
        g = base + c * _CH
        l0 = lax.rem(g, _S)
        pltpu.async_copy(
            table_hbm.at[idx_v.at[pl.ds(c * _CH, _CH)]], buf, sem
        ).wait()

        def row_body(r, carry2):
            pr = l0 + r
            for k in range(_D // _LANES):
                sl = pl.ds(k * _LANES, _LANES)
                buf[r, sl] = buf[r, sl] + pe2_v[pr, sl]
            return carry2

        lax.fori_loop(0, _CH, row_body, 0)
        pltpu.sync_copy(buf, out_hbm.at[pl.ds(g, _CH)])
        return carry

    lax.fori_loop(0, _CHUNKS, chunk_body, 0)


def kernel(x, table):
    idx = x.reshape(-1).astype(jnp.int32)
    out = _emb_pe(idx, table, _PE2)
    return out.reshape(_B, _S, _D)


# speedup vs baseline: 1.0517x; 1.0517x over previous
import functools

import numpy as np
import jax
import jax.numpy as jnp
from jax import lax
from jax.experimental import pallas as pl
from jax.experimental.pallas import tpu as pltpu
from jax.experimental.pallas import tpu_sc as plsc

_B = 1024
_S = 200
_D = 64
_N = _B * _S
_NC = 2
_NS = 16
_NW = _NC * _NS
_PER_W = _N // _NW
_CH = 128
_NBUF = 5
_OUTER = _PER_W // (_CH * _NBUF)
_LANES = 16


def _pe_doubled() -> np.ndarray:
    pos = np.arange(_S, dtype=np.float32)[:, None]
    div = 10000.0 ** (np.arange(0, _D, 2, dtype=np.float32) / _D)
    pe = np.zeros((_S, _D), dtype=np.float32)
    pe[:, 0::2] = np.sin(pos / div)
    pe[:, 1::2] = np.cos(pos / div)
    return np.concatenate([pe, pe], axis=0)


_PE2 = _pe_doubled()

_mesh = plsc.VectorSubcoreMesh(core_axis_name="c", subcore_axis_name="s")


@functools.partial(
    pl.kernel,
    mesh=_mesh,
    out_type=jax.ShapeDtypeStruct((_N, _D), jnp.float32),
    compiler_params=pltpu.CompilerParams(use_tc_tiling_on_sc=False),
    scratch_types=(
        [pltpu.VMEM((_PER_W,), jnp.int32),
         pltpu.VMEM((2 * _S, _D), jnp.float32)]
        + [pltpu.VMEM((_CH, _D), jnp.float32) for _ in range(_NBUF)]
        + [pltpu.SemaphoreType.DMA for _ in range(2 * _NBUF)]
    ),
)
def _emb_pe(idx_hbm, table_hbm, pe2_hbm, out_hbm, idx_v, pe2_v, *rest):
    bufs = rest[:_NBUF]
    gsems = rest[_NBUF:2 * _NBUF]
    wsems = rest[2 * _NBUF:]

    wid = lax.axis_index("s") * _NC + lax.axis_index("c")
    base = wid * _PER_W
    pltpu.sync_copy(idx_hbm.at[pl.ds(base, _PER_W)], idx_v)
    pltpu.sync_copy(pe2_hbm, pe2_v)

    def outer(cc, carry):
        c0 = cc * _NBUF

        for b in range(_NBUF):
            c = c0 + b
            g = base + c * _CH

            @pl.when(cc > 0)
            def _():
                pltpu.make_async_copy(
                    bufs[b], out_hbm.at[pl.ds(g, _CH)], wsems[b]
                ).wait()

            pltpu.async_copy(
                table_hbm.at[idx_v.at[pl.ds(c * _CH, _CH)]], bufs[b], gsems[b]
            )

        for b in range(_NBUF):
            c = c0 + b
            g = base + c * _CH
            l0 = lax.rem(g, _S)
            pltpu.make_async_copy(
                table_hbm.at[idx_v.at[pl.ds(c * _CH, _CH)]], bufs[b], gsems[b]
            ).wait()

            buf = bufs[b]

            def row_body(r, carry2, buf=buf, l0=l0):
                pr = l0 + r
                for k in range(_D // _LANES):
                    sl = pl.ds(k * _LANES, _LANES)
                    buf[r, sl] = buf[r, sl] + pe2_v[pr, sl]
                return carry2

            lax.fori_loop(0, _CH, row_body, 0)
            pltpu.async_copy(buf, out_hbm.at[pl.ds(g, _CH)], wsems[b])
        return carry

    lax.fori_loop(0, _OUTER, outer, 0)

    for b in range(_NBUF):
        c = (_OUTER - 1) * _NBUF + b
        g = base + c * _CH
        pltpu.make_async_copy(
            bufs[b], out_hbm.at[pl.ds(g, _CH)], wsems[b]
        ).wait()


def kernel(x, table):
    idx = x.reshape(-1).astype(jnp.int32)
    out = _emb_pe(idx, table, _PE2)
    return out.reshape(_B, _S, _D)


# speedup vs baseline: 1.1930x; 1.1344x over previous
import functools

import numpy as np
import jax
import jax.numpy as jnp
from jax import lax
from jax.experimental import pallas as pl
from jax.experimental.pallas import tpu as pltpu
from jax.experimental.pallas import tpu_sc as plsc

_B = 1024
_S = 200
_D = 64
_N = _B * _S
_NC = 2
_NS = 16
_NW = _NC * _NS
_SEQ_W = _B // _NW
_C1 = 128
_C2 = _S - _C1
_NBUF = 4
_OUTER = _SEQ_W // _NBUF
_LANES = 16


def _pe_table() -> np.ndarray:
    pos = np.arange(_S, dtype=np.float32)[:, None]
    div = 10000.0 ** (np.arange(0, _D, 2, dtype=np.float32) / _D)
    pe = np.zeros((_S, _D), dtype=np.float32)
    pe[:, 0::2] = np.sin(pos / div)
    pe[:, 1::2] = np.cos(pos / div)
    return pe


_PE = _pe_table()

_mesh = plsc.VectorSubcoreMesh(core_axis_name="c", subcore_axis_name="s")


@functools.partial(
    pl.kernel,
    mesh=_mesh,
    out_type=jax.ShapeDtypeStruct((_B, _S, _D), jnp.float32),
    compiler_params=pltpu.CompilerParams(use_tc_tiling_on_sc=False),
    scratch_types=(
        [pltpu.VMEM((_SEQ_W * _S,), jnp.int32),
         pltpu.VMEM((_S, _D), jnp.float32)]
        + [pltpu.VMEM((_S, _D), jnp.float32) for _ in range(_NBUF)]
        + [pltpu.SemaphoreType.DMA for _ in range(2 * _NBUF)]
    ),
)
def _emb_pe(idx_hbm, table_hbm, pe_hbm, out_hbm, idx_v, pe_v, *rest):
    bufs = rest[:_NBUF]
    gsems = rest[_NBUF:2 * _NBUF]
    wsems = rest[2 * _NBUF:]

    wid = lax.axis_index("s") * _NC + lax.axis_index("c")
    sbase = wid * _SEQ_W
    pltpu.sync_copy(idx_hbm.at[pl.ds(sbase * _S, _SEQ_W * _S)], idx_v)
    pltpu.sync_copy(pe_hbm, pe_v)

    def gather(s, buf, gsem):
        pltpu.async_copy(
            table_hbm.at[idx_v.at[pl.ds(s * _S, _C1)]],
            buf.at[pl.ds(0, _C1)], gsem,
        )
        pltpu.async_copy(
            table_hbm.at[idx_v.at[pl.ds(s * _S + _C1, _C2)]],
            buf.at[pl.ds(_C1, _C2)], gsem,
        )

    def wait_gather(s, buf, gsem):
        pltpu.make_async_copy(
            table_hbm.at[idx_v.at[pl.ds(s * _S, _C1)]],
            buf.at[pl.ds(0, _C1)], gsem,
        ).wait()
        pltpu.make_async_copy(
            table_hbm.at[idx_v.at[pl.ds(s * _S + _C1, _C2)]],
            buf.at[pl.ds(_C1, _C2)], gsem,
        ).wait()

    def outer(cc, carry):
        s0 = cc * _NBUF
        for b in range(_NBUF):
            s = s0 + b

            @pl.when(cc > 0)
            def _():
                pltpu.make_async_copy(
                    bufs[b], out_hbm.at[sbase + s], wsems[b]
                ).wait()

            gather(s, bufs[b], gsems[b])

        for b in range(_NBUF):
            s = s0 + b
            wait_gather(s, bufs[b], gsems[b])
            buf = bufs[b]

            def row_body(r, carry2, buf=buf):
                for k in range(_D // _LANES):
                    sl = pl.ds(k * _LANES, _LANES)
                    buf[r, sl] = buf[r, sl] + pe_v[r, sl]
                return carry2

            lax.fori_loop(0, _S, row_body, 0)
            pltpu.async_copy(buf, out_hbm.at[sbase + s], wsems[b])
        return carry

    lax.fori_loop(0, _OUTER, outer, 0)

    for b in range(_NBUF):
        s = (_OUTER - 1) * _NBUF + b
        pltpu.make_async_copy(
            bufs[b], out_hbm.at[sbase + s], wsems[b]
        ).wait()


def kernel(x, table):
    idx = x.reshape(-1).astype(jnp.int32)
    return _emb_pe(idx, table, _PE)
